# Initial kernel scaffold; baseline (speedup 1.0000x reference)
import jax
import jax.numpy as jnp
from jax.experimental import pallas as pl


def kernel(x, edge_index, W1_l, W1_r, b1, W2_l, W2_r, b2, Wc, bc):
    raise NotImplementedError("write your pallas kernel here")


# speedup vs baseline: 9.1244x; 9.1244x over previous
import functools

import jax
import jax.numpy as jnp
from jax import lax
from jax.experimental import pallas as pl
from jax.experimental.pallas import tpu as pltpu
from jax.experimental.pallas import tpu_sc as plsc

_NC = 2
_NS = 16
_CH = 128
_IB = 80
_ZR = 256


def _make_seg_sum(n_out_pad, table_rows, src_rows, per_core_rows,
                  rows_per_tile, src_core_stride):
  n_blocks = rows_per_tile // _IB
  nacc = n_out_pad
  zspan = nacc // _NS
  zfills = zspan // _ZR
  zrem = zspan - zfills * _ZR
  out_rows = n_out_pad // _NS
  mesh = plsc.VectorSubcoreMesh(core_axis_name="c", subcore_axis_name="s")

  @functools.partial(
      pl.kernel,
      out_type=jax.ShapeDtypeStruct((_NC, n_out_pad, 16), jnp.float32),
      mesh=mesh,
      compiler_params=pltpu.CompilerParams(use_tc_tiling_on_sc=False),
      scratch_types=[
          pltpu.VMEM((_ZR, 16), jnp.float32),
          pltpu.VMEM((_IB, _CH), jnp.int32),
          pltpu.VMEM((_IB, _CH), jnp.int32),
          pltpu.VMEM((_CH, 16), jnp.float32),
          pltpu.VMEM_SHARED((nacc, 16), jnp.float32),
          pltpu.SemaphoreType.DMA,
      ],
  )
  def seg(table_hbm, src_hbm, dst_hbm, out_hbm, zbuf, sbuf, dbuf, rbuf, acc,
          sem):
    c = lax.axis_index("c")
    s = lax.axis_index("s")

    def zfill(i, carry):
      zbuf[i] = jnp.zeros((16,), jnp.float32)
      return carry
    lax.fori_loop(0, _ZR, zfill, 0)

    zbase = s * zspan

    def zcopy(k, carry):
      pltpu.sync_copy(zbuf, acc.at[pl.ds(zbase + k * _ZR, _ZR)])
      return carry
    lax.fori_loop(0, zfills, zcopy, 0)
    if zrem:
      pltpu.sync_copy(zbuf.at[pl.ds(0, zrem)],
                      acc.at[pl.ds(zbase + zfills * _ZR, zrem)])
    plsc.subcore_barrier()

    dst_base = c * per_core_rows + s * rows_per_tile
    src_base = c * src_core_stride + dst_base

    def blk(b, carry):
      pltpu.sync_copy(src_hbm.at[pl.ds(src_base + b * _IB, _IB)], sbuf)
      pltpu.sync_copy(dst_hbm.at[pl.ds(dst_base + b * _IB, _IB)], dbuf)

      def inner(j, icarry):
        pltpu.async_copy(table_hbm.at[sbuf.at[j]], rbuf, sem).wait()
        pltpu.sync_copy(rbuf, acc.at[dbuf.at[j]], add=True)
        return icarry
      lax.fori_loop(0, _IB, inner, 0)
      return carry
    lax.fori_loop(0, n_blocks, blk, 0)
    plsc.subcore_barrier()

    ob = s * out_rows
    pltpu.sync_copy(acc.at[pl.ds(ob, out_rows)],
                    out_hbm.at[c, pl.ds(ob, out_rows)])

  return seg


def _dense1(agg1, x, W1_l, W1_r, b1, W2_l, W2_r, b2):
  n = x.shape[0]
  r = 2000
  grid = (n // r,)

  def body(a_ref, x_ref, w1l_ref, w1r_ref, b1_ref, w2l_ref, w2r_ref, b2_ref,
           hw_ref, hr_ref, inv_ref):
    a = a_ref[0] + a_ref[1]
    inv = 1.0 / jnp.maximum(a[:, 11:12], 1.0)
    mean1 = a[:, :11] * inv
    h = jnp.maximum(
        jnp.dot(mean1, w1l_ref[...], preferred_element_type=jnp.float32)
        + jnp.dot(x_ref[...], w1r_ref[...], preferred_element_type=jnp.float32)
        + b1_ref[...], 0.0)
    hl = jnp.dot(h, w2l_ref[...], preferred_element_type=jnp.float32)
    hw_ref[0] = hl[:, :16]
    hw_ref[1] = hl[:, 16:]
    hr_ref[...] = (jnp.dot(h, w2r_ref[...], preferred_element_type=jnp.float32)
                   + b2_ref[...])
    inv_ref[...] = inv

  return pl.pallas_call(
      body,
      grid=grid,
      in_specs=[
          pl.BlockSpec((2, r, 16), lambda i: (0, i, 0)),
          pl.BlockSpec((r, 11), lambda i: (i, 0)),
          pl.BlockSpec((11, 64), lambda i: (0, 0)),
          pl.BlockSpec((11, 64), lambda i: (0, 0)),
          pl.BlockSpec((1, 64), lambda i: (0, 0)),
          pl.BlockSpec((64, 32), lambda i: (0, 0)),
          pl.BlockSpec((64, 32), lambda i: (0, 0)),
          pl.BlockSpec((1, 32), lambda i: (0, 0)),
      ],
      out_specs=[
          pl.BlockSpec((2, r, 16), lambda i: (0, i, 0)),
          pl.BlockSpec((r, 32), lambda i: (i, 0)),
          pl.BlockSpec((r, 1), lambda i: (i, 0)),
      ],
      out_shape=[
          jax.ShapeDtypeStruct((2, n, 16), jnp.float32),
          jax.ShapeDtypeStruct((n, 32), jnp.float32),
          jax.ShapeDtypeStruct((n, 1), jnp.float32),
      ],
  )(agg1, x, W1_l, W1_r, b1.reshape(1, -1), W2_l, W2_r, b2.reshape(1, -1))


def _dense2(agg2, inv, hr, Wc, bc):
  n = hr.shape[0]
  r = 2000
  grid = (n // r,)

  def body(a_ref, inv_ref, hr_ref, wc_ref, bc_ref, logits_ref, emb_ref):
    a = jnp.concatenate([a_ref[0], a_ref[1]], axis=1)
    emb = jnp.maximum(a * inv_ref[...] + hr_ref[...], 0.0)
    emb_ref[...] = emb
    logits_ref[...] = (jnp.dot(emb, wc_ref[...],
                               preferred_element_type=jnp.float32)
                       + bc_ref[...])

  return pl.pallas_call(
      body,
      grid=grid,
      in_specs=[
          pl.BlockSpec((2, r, 16), lambda i: (0, i, 0)),
          pl.BlockSpec((r, 1), lambda i: (i, 0)),
          pl.BlockSpec((r, 32), lambda i: (i, 0)),
          pl.BlockSpec((32, 3), lambda i: (0, 0)),
          pl.BlockSpec((1, 3), lambda i: (0, 0)),
      ],
      out_specs=[
          pl.BlockSpec((r, 3), lambda i: (i, 0)),
          pl.BlockSpec((r, 32), lambda i: (i, 0)),
      ],
      out_shape=[
          jax.ShapeDtypeStruct((n, 3), jnp.float32),
          jax.ShapeDtypeStruct((n, 32), jnp.float32),
      ],
  )(agg2, inv, hr, Wc, bc.reshape(1, -1))


def kernel(x, edge_index, W1_l, W1_r, b1, W2_l, W2_r, b2, Wc, bc):
  n = x.shape[0]
  e = edge_index.shape[1]

  row_quant = _CH * _NC * _NS * _IB
  ep = ((e + row_quant - 1) // row_quant) * row_quant
  rows = ep // _CH
  src = edge_index[0].astype(jnp.int32)
  dst = edge_index[1].astype(jnp.int32)
  src_p = jnp.concatenate(
      [src, jnp.zeros((ep - e,), jnp.int32)]).reshape(rows, _CH)
  dst_p = jnp.concatenate(
      [dst, jnp.full((ep - e,), n, jnp.int32)]).reshape(rows, _CH)

  np8 = 8 * _NS
  n_out_pad = ((n + np8) // np8) * np8

  xp = jnp.concatenate(
      [x, jnp.ones((n, 1), x.dtype), jnp.zeros((n, 4), x.dtype)], axis=1)
  seg1 = _make_seg_sum(
      n_out_pad=n_out_pad, table_rows=n, src_rows=rows,
      per_core_rows=rows // _NC, rows_per_tile=rows // (_NC * _NS),
      src_core_stride=0)
  agg1 = seg1(xp, src_p, dst_p)

  hw, hr, inv = _dense1(agg1, x, W1_l, W1_r, b1, W2_l, W2_r, b2)

  table2 = hw.reshape(2 * n, 16)
  src2 = jnp.concatenate([src_p, src_p + n], axis=0)
  seg2 = _make_seg_sum(
      n_out_pad=n_out_pad, table_rows=2 * n, src_rows=2 * rows,
      per_core_rows=0, rows_per_tile=rows // _NS,
      src_core_stride=rows)
  agg2 = seg2(table2, src2, dst_p)

  logits, emb = _dense2(agg2, inv, hr, Wc, bc)
  return (logits, emb)


# speedup vs baseline: 13.7342x; 1.5052x over previous
import functools

import jax
import jax.numpy as jnp
from jax import lax
from jax.experimental import pallas as pl
from jax.experimental.pallas import tpu as pltpu
from jax.experimental.pallas import tpu_sc as plsc

_NC = 2
_NS = 16
_CH = 128
_IB = 40
_ZR = 128
_RS = 8


def _make_seg_sum(n_out_pad, table_rows, src_rows, per_core_rows,
                  rows_per_tile, src_core_stride):
  n_blocks = rows_per_tile // _IB
  nacc = n_out_pad
  zspan = nacc // _NS
  zfills = zspan // _ZR
  zrem = zspan - zfills * _ZR
  out_rows = n_out_pad // _NS
  mesh = plsc.VectorSubcoreMesh(core_axis_name="c", subcore_axis_name="s")

  @functools.partial(
      pl.kernel,
      out_type=jax.ShapeDtypeStruct((_NC, n_out_pad, 16), jnp.float32),
      mesh=mesh,
      compiler_params=pltpu.CompilerParams(use_tc_tiling_on_sc=False),
      scratch_types=[
          pltpu.VMEM((_ZR, 16), jnp.float32),
          pltpu.VMEM((_IB, _CH), jnp.int32),
          pltpu.VMEM((_IB, _CH), jnp.int32),
          pltpu.VMEM((_RS * _CH, 16), jnp.float32),
          pltpu.VMEM_SHARED((nacc, 16), jnp.float32),
          pltpu.SemaphoreType.DMA((_RS,)),
          pltpu.SemaphoreType.DMA((_RS,)),
          pltpu.SemaphoreType.DMA,
      ],
  )
  def seg(table_hbm, src_hbm, dst_hbm, out_hbm, zbuf, sbuf, dbuf, slab, acc,
          gsem, ssem, zsem):
    c = lax.axis_index("c")
    s = lax.axis_index("s")

    def slot(b):
      return slab.at[pl.ds(b * _CH, _CH)]

    def zfill(i, carry):
      zbuf[i] = jnp.zeros((16,), jnp.float32)
      return carry
    lax.fori_loop(0, _ZR, zfill, 0)

    zbase = s * zspan

    def zcopy(k, carry):
      for b in range(8):
        pltpu.async_copy(
            zbuf, acc.at[pl.ds(zbase + (k * 8 + b) * _ZR, _ZR)], zsem)
      for b in range(8):
        pltpu.make_async_copy(
            zbuf, acc.at[pl.ds(zbase + (k * 8 + b) * _ZR, _ZR)], zsem).wait()
      return carry
    lax.fori_loop(0, zfills // 8, zcopy, 0)
    for k in range(zfills - (zfills // 8) * 8):
      pltpu.sync_copy(zbuf,
                      acc.at[pl.ds(zbase + ((zfills // 8) * 8 + k) * _ZR,
                                   _ZR)])
    if zrem:
      pltpu.sync_copy(zbuf.at[pl.ds(0, zrem)],
                      acc.at[pl.ds(zbase + zfills * _ZR, zrem)])
    plsc.subcore_barrier()

    dst_base = c * per_core_rows + s * rows_per_tile
    src_base = c * src_core_stride + dst_base

    def chunk(ci, carry):
      pltpu.sync_copy(src_hbm.at[pl.ds(src_base + ci * _IB, _IB)], sbuf)
      pltpu.sync_copy(dst_hbm.at[pl.ds(dst_base + ci * _IB, _IB)], dbuf)

      def gpair(gp, carry2):
        for gg in range(2):
          for i in range(4):
            b = 4 * gg + i
            lr = gp * 8 + b

            @pl.when(gp > 0)
            def _wait_scatter(b=b, lr=lr):
              pltpu.make_async_copy(
                  slot(b), acc.at[dbuf.at[lr - 8]], ssem.at[b]).wait()
            pltpu.async_copy(table_hbm.at[sbuf.at[lr]], slot(b), gsem.at[b])
          for i in range(4):
            pb = 4 * (1 - gg) + i
            pr = gp * 8 + 4 * gg - 4 + i

            def _scat(pb=pb, pr=pr):
              pltpu.make_async_copy(
                  table_hbm.at[sbuf.at[pr]], slot(pb), gsem.at[pb]).wait()
              pltpu.async_copy(
                  slot(pb), acc.at[dbuf.at[pr]], ssem.at[pb], add=True)
            if gg == 0:
              pl.when(gp > 0)(_scat)
            else:
              _scat()
        return carry2
      lax.fori_loop(0, _IB // 8, gpair, 0)

      for i in range(4):
        pb = 4 + i
        pr = _IB - 4 + i
        pltpu.make_async_copy(
            table_hbm.at[sbuf.at[pr]], slot(pb), gsem.at[pb]).wait()
        pltpu.async_copy(slot(pb), acc.at[dbuf.at[pr]], ssem.at[pb], add=True)
      for b in range(8):
        pltpu.make_async_copy(
            slot(b), acc.at[dbuf.at[_IB - 8 + b]], ssem.at[b]).wait()
      return carry
    lax.fori_loop(0, n_blocks, chunk, 0)
    plsc.subcore_barrier()

    ob = s * out_rows
    pltpu.sync_copy(acc.at[pl.ds(ob, out_rows)],
                    out_hbm.at[c, pl.ds(ob, out_rows)])

  return seg


def _dense1(agg1, x, W1_l, W1_r, b1, W2_l, W2_r, b2):
  n = x.shape[0]
  r = 2000
  grid = (n // r,)

  def body(a_ref, x_ref, w1l_ref, w1r_ref, b1_ref, w2l_ref, w2r_ref, b2_ref,
           hw_ref, hr_ref, inv_ref):
    a = a_ref[0] + a_ref[1]
    inv = 1.0 / jnp.maximum(a[:, 11:12], 1.0)
    mean1 = a[:, :11] * inv
    h = jnp.maximum(
        jnp.dot(mean1, w1l_ref[...], preferred_element_type=jnp.float32)
        + jnp.dot(x_ref[...], w1r_ref[...], preferred_element_type=jnp.float32)
        + b1_ref[...], 0.0)
    hl = jnp.dot(h, w2l_ref[...], preferred_element_type=jnp.float32)
    hw_ref[0] = hl[:, :16]
    hw_ref[1] = hl[:, 16:]
    hr_ref[...] = (jnp.dot(h, w2r_ref[...], preferred_element_type=jnp.float32)
                   + b2_ref[...])
    inv_ref[...] = inv

  return pl.pallas_call(
      body,
      grid=grid,
      in_specs=[
          pl.BlockSpec((2, r, 16), lambda i: (0, i, 0)),
          pl.BlockSpec((r, 11), lambda i: (i, 0)),
          pl.BlockSpec((11, 64), lambda i: (0, 0)),
          pl.BlockSpec((11, 64), lambda i: (0, 0)),
          pl.BlockSpec((1, 64), lambda i: (0, 0)),
          pl.BlockSpec((64, 32), lambda i: (0, 0)),
          pl.BlockSpec((64, 32), lambda i: (0, 0)),
          pl.BlockSpec((1, 32), lambda i: (0, 0)),
      ],
      out_specs=[
          pl.BlockSpec((2, r, 16), lambda i: (0, i, 0)),
          pl.BlockSpec((r, 32), lambda i: (i, 0)),
          pl.BlockSpec((r, 1), lambda i: (i, 0)),
      ],
      out_shape=[
          jax.ShapeDtypeStruct((2, n, 16), jnp.float32),
          jax.ShapeDtypeStruct((n, 32), jnp.float32),
          jax.ShapeDtypeStruct((n, 1), jnp.float32),
      ],
  )(agg1, x, W1_l, W1_r, b1.reshape(1, -1), W2_l, W2_r, b2.reshape(1, -1))


def _dense2(agg2, inv, hr, Wc, bc):
  n = hr.shape[0]
  r = 2000
  grid = (n // r,)

  def body(a_ref, inv_ref, hr_ref, wc_ref, bc_ref, logits_ref, emb_ref):
    a = jnp.concatenate([a_ref[0], a_ref[1]], axis=1)
    emb = jnp.maximum(a * inv_ref[...] + hr_ref[...], 0.0)
    emb_ref[...] = emb
    logits_ref[...] = (jnp.dot(emb, wc_ref[...],
                               preferred_element_type=jnp.float32)
                       + bc_ref[...])

  return pl.pallas_call(
      body,
      grid=grid,
      in_specs=[
          pl.BlockSpec((2, r, 16), lambda i: (0, i, 0)),
          pl.BlockSpec((r, 1), lambda i: (i, 0)),
          pl.BlockSpec((r, 32), lambda i: (i, 0)),
          pl.BlockSpec((32, 3), lambda i: (0, 0)),
          pl.BlockSpec((1, 3), lambda i: (0, 0)),
      ],
      out_specs=[
          pl.BlockSpec((r, 3), lambda i: (i, 0)),
          pl.BlockSpec((r, 32), lambda i: (i, 0)),
      ],
      out_shape=[
          jax.ShapeDtypeStruct((n, 3), jnp.float32),
          jax.ShapeDtypeStruct((n, 32), jnp.float32),
      ],
  )(agg2, inv, hr, Wc, bc.reshape(1, -1))


def kernel(x, edge_index, W1_l, W1_r, b1, W2_l, W2_r, b2, Wc, bc):
  n = x.shape[0]
  e = edge_index.shape[1]

  row_quant = _CH * _NC * _NS * _IB
  ep = ((e + row_quant - 1) // row_quant) * row_quant
  rows = ep // _CH
  src = edge_index[0].astype(jnp.int32)
  dst = edge_index[1].astype(jnp.int32)
  src_p = jnp.concatenate(
      [src, jnp.zeros((ep - e,), jnp.int32)]).reshape(rows, _CH)
  dst_p = jnp.concatenate(
      [dst, jnp.full((ep - e,), n, jnp.int32)]).reshape(rows, _CH)

  np8 = 8 * _NS
  n_out_pad = ((n + np8) // np8) * np8

  xp = jnp.concatenate(
      [x, jnp.ones((n, 1), x.dtype), jnp.zeros((n, 4), x.dtype)], axis=1)
  seg1 = _make_seg_sum(
      n_out_pad=n_out_pad, table_rows=n, src_rows=rows,
      per_core_rows=rows // _NC, rows_per_tile=rows // (_NC * _NS),
      src_core_stride=0)
  agg1 = seg1(xp, src_p, dst_p)

  hw, hr, inv = _dense1(agg1, x, W1_l, W1_r, b1, W2_l, W2_r, b2)

  table2 = hw.reshape(2 * n, 16)
  src2 = jnp.concatenate([src_p, src_p + n], axis=0)
  seg2 = _make_seg_sum(
      n_out_pad=n_out_pad, table_rows=2 * n, src_rows=2 * rows,
      per_core_rows=0, rows_per_tile=rows // _NS,
      src_core_stride=rows)
  agg2 = seg2(table2, src2, dst_p)

  logits, emb = _dense2(agg2, inv, hr, Wc, bc)
  return (logits, emb)
